# baseline (speedup 1.0000x reference)
import functools

import jax
import jax.numpy as jnp
from jax import lax
from jax.experimental import pallas as pl
from jax.experimental.pallas import tpu as pltpu
from jax.experimental.pallas import tpu_sc as plsc

N = 10000
E = 320000
D = 128

NC = 2
NS = 16
L = 16
TILES = NC * NS

E2 = E + N
EPT = 10752
E_PAD = EPT * TILES
CH = 112
NCHUNK = EPT // CH
NTRIP = NCHUNK // 3
NP = 10240
RPT = NP // NS


def _deg_body(dst_hbm, w_hbm, out_hbm, deg_l, dst_all, w_all):
    c = lax.axis_index("c")
    s = lax.axis_index("s")
    wid = c * NS + s

    def zbody(i, _):
        deg_l[pl.ds(i * L, L)] = jnp.zeros((L,), jnp.float32)
        return 0

    lax.fori_loop(0, N // L, zbody, 0)

    base = wid * EPT
    pltpu.sync_copy(dst_hbm.at[pl.ds(base, EPT)], dst_all)
    pltpu.sync_copy(w_hbm.at[pl.ds(base, EPT)], w_all)

    def vec_body(i, _):
        sl = pl.ds(i * L, L)
        plsc.addupdate_scatter(deg_l, [dst_all[sl]], w_all[sl])
        return 0

    lax.fori_loop(0, EPT // L, vec_body, 0)
    pltpu.sync_copy(deg_l, out_hbm.at[wid])


def _agg_body(packed_hbm, h_hbm, zeros_hbm, out_hbm,
              bank0, bank1, bank2, rows0, rows1, rows2, acc_sh,
              gs0, gs1, gs2, ss0, ss1, ss2):
    c = lax.axis_index("c")
    s = lax.axis_index("s")
    wid = c * NS + s
    bank = (bank0, bank1, bank2)
    rows = (rows0, rows1, rows2)
    gs = (gs0, gs1, gs2)
    ss = (ss0, ss1, ss2)

    rbase = wid * NCHUNK
    pltpu.sync_copy(zeros_hbm.at[pl.ds(s * RPT, RPT)],
                    acc_sh.at[pl.ds(s * RPT, RPT)])
    plsc.subcore_barrier()

    def stage_and_gather(ci, k):
        pltpu.sync_copy(packed_hbm.at[rbase + ci], bank[k])
        pltpu.async_copy(h_hbm.at[bank[k].at[0]], rows[k], gs[k])

    stage_and_gather(0, 0)
    stage_and_gather(1, 1)

    def process(i, k):
        pltpu.make_async_copy(h_hbm.at[bank[k].at[0]], rows[k], gs[k]).wait()
        rk = rows[k]
        bk = bank[k]

        def scale_group(g, _):
            wv = plsc.bitcast(bk[2, pl.ds(g * L, L)], jnp.float32)
            for t in range(L):
                nv = wv[t]
                e = g * L + t
                for j in range(D // L):
                    sl2 = pl.ds(j * L, L)
                    rk[e, sl2] = rk[e, sl2] * nv
            return 0

        lax.fori_loop(0, CH // L, scale_group, 0)
        pltpu.async_copy(rk, acc_sh.at[bk.at[1]], ss[k], add=True)
        nk = (k + 2) % 3

        @pl.when(i + 2 < NCHUNK)
        def _():
            @pl.when(i >= 1)
            def _():
                pltpu.make_async_copy(rows[nk], acc_sh.at[bank[nk].at[1]],
                                      ss[nk]).wait()
            stage_and_gather(i + 2, nk)

    def triple_body(tr, _):
        process(3 * tr, 0)
        process(3 * tr + 1, 1)
        process(3 * tr + 2, 2)
        return 0

    lax.fori_loop(0, NTRIP, triple_body, 0)
    for k in range(3):
        pltpu.make_async_copy(rows[k], acc_sh.at[bank[k].at[1]], ss[k]).wait()
    plsc.subcore_barrier()
    pltpu.sync_copy(acc_sh.at[pl.ds(s * RPT, RPT)],
                    out_hbm.at[c, pl.ds(s * RPT, RPT)])


@functools.lru_cache(maxsize=None)
def _sc_kernels():
    mesh = plsc.VectorSubcoreMesh(core_axis_name="c", subcore_axis_name="s")
    deg_k = pl.kernel(
        _deg_body,
        out_type=jax.ShapeDtypeStruct((TILES, N), jnp.float32),
        mesh=mesh,
        scratch_types=[
            pltpu.VMEM((N,), jnp.float32),
            pltpu.VMEM((EPT,), jnp.int32),
            pltpu.VMEM((EPT,), jnp.float32),
        ],
        compiler_params=pltpu.CompilerParams(needs_layout_passes=False),
    )
    agg_k = pl.kernel(
        _agg_body,
        out_type=jax.ShapeDtypeStruct((NC, NP, D), jnp.float32),
        mesh=mesh,
        scratch_types=[
            pltpu.VMEM((3, CH), jnp.int32),
            pltpu.VMEM((3, CH), jnp.int32),
            pltpu.VMEM((3, CH), jnp.int32),
            pltpu.VMEM((CH, D), jnp.float32),
            pltpu.VMEM((CH, D), jnp.float32),
            pltpu.VMEM((CH, D), jnp.float32),
            pltpu.VMEM_SHARED((NP, D), jnp.float32),
            pltpu.SemaphoreType.DMA,
            pltpu.SemaphoreType.DMA,
            pltpu.SemaphoreType.DMA,
            pltpu.SemaphoreType.DMA,
            pltpu.SemaphoreType.DMA,
            pltpu.SemaphoreType.DMA,
        ],
        compiler_params=pltpu.CompilerParams(needs_layout_passes=False),
    )
    return deg_k, agg_k


def _tc1_body(degp_ref, x_ref, w1_ref, dinv_ref, h1_ref):
    deg = jnp.sum(degp_ref[...], axis=0, keepdims=True)
    dinv_ref[...] = jnp.where(deg > 0, lax.rsqrt(deg), 0.0)
    h1_ref[...] = jnp.dot(x_ref[...], w1_ref[...],
                          preferred_element_type=jnp.float32)


def _bn_relu(accp_ref, dinv_col_ref, b_ref, g_ref, be_ref):
    dc = dinv_col_ref[...]
    a = dc * (accp_ref[0, :N, :] + accp_ref[1, :N, :]) + b_ref[...]
    a = jnp.maximum(a, 0.0)
    m = jnp.mean(a, axis=0, keepdims=True)
    v = jnp.maximum(jnp.mean(a * a, axis=0, keepdims=True) - m * m, 0.0)
    return g_ref[...] * (a - m) * lax.rsqrt(v + 1e-5) + be_ref[...]


def _tc_mid_body(accp_ref, dinv_col_ref, b_ref, g_ref, be_ref, w2_ref,
                 h2_ref):
    h = _bn_relu(accp_ref, dinv_col_ref, b_ref, g_ref, be_ref)
    h2_ref[...] = dinv_col_ref[...] * jnp.dot(
        h, w2_ref[...], preferred_element_type=jnp.float32)


def _tc_post_body(accp_ref, dinv_col_ref, b_ref, g_ref, be_ref, out_ref):
    out_ref[...] = _bn_relu(accp_ref, dinv_col_ref, b_ref, g_ref, be_ref)


_tc1 = pl.pallas_call(
    _tc1_body,
    out_shape=[jax.ShapeDtypeStruct((1, N), jnp.float32),
               jax.ShapeDtypeStruct((N, D), jnp.float32)],
)

_tc_mid = pl.pallas_call(
    _tc_mid_body,
    out_shape=jax.ShapeDtypeStruct((N, D), jnp.float32),
)

_tc_post = pl.pallas_call(
    _tc_post_body,
    out_shape=jax.ShapeDtypeStruct((N, D), jnp.float32),
)


def kernel(x, edge_index, edge_attr, W1, b1, g1, be1, W2, b2, g2, be2):
    src = edge_index[0]
    dst = edge_index[1]
    loop = jnp.arange(N, dtype=jnp.int32)
    padn = E_PAD - E2
    zi = jnp.zeros((padn,), jnp.int32)
    src2 = jnp.concatenate([src, loop, zi])
    dst2 = jnp.concatenate([dst, loop, zi])
    w2_ = jnp.concatenate([edge_attr, jnp.ones((N,), jnp.float32),
                           jnp.zeros((padn,), jnp.float32)])

    packed = jnp.stack([
        src2.reshape(TILES * NCHUNK, CH),
        dst2.reshape(TILES * NCHUNK, CH),
        jax.lax.bitcast_convert_type(w2_, jnp.int32).reshape(
            TILES * NCHUNK, CH),
    ], axis=1)

    deg_k, agg_k = _sc_kernels()
    degp = deg_k(dst2, w2_)
    dinv2d, hw = _tc1(degp, x, W1)
    dinv_col = dinv2d.reshape(N, 1)
    h1p = dinv_col * hw
    zeros_nd = jnp.zeros((NP, D), jnp.float32)

    accp1 = agg_k(packed, h1p, zeros_nd)
    h2p = _tc_mid(accp1, dinv_col, b1.reshape(1, D), g1.reshape(1, D),
                  be1.reshape(1, D), W2)
    accp2 = agg_k(packed, h2p, zeros_nd)
    out = _tc_post(accp2, dinv_col, b2.reshape(1, D), g2.reshape(1, D),
                   be2.reshape(1, D))
    return out


# speedup vs baseline: 3.1536x; 3.1536x over previous
import functools

import jax
import jax.numpy as jnp
from jax import lax
from jax.experimental import pallas as pl
from jax.experimental.pallas import tpu as pltpu
from jax.experimental.pallas import tpu_sc as plsc

N = 10000
E = 320000
D = 128

NC = 2
NS = 16
L = 16
TILES = NC * NS

E2 = E + N
EPT = 10368
E_PAD = EPT * TILES
CH = 96
NCHUNK = EPT // CH
NTRIP = NCHUNK // 3
NP = 10240
RPT = NP // NS


def _deg_body(dst_hbm, w_hbm, out_hbm, deg_l, dst_all, w_all):
    c = lax.axis_index("c")
    s = lax.axis_index("s")
    wid = c * NS + s

    def zbody(i, _):
        deg_l[pl.ds(i * L, L)] = jnp.zeros((L,), jnp.float32)
        return 0

    lax.fori_loop(0, NP // L, zbody, 0)

    base = wid * EPT
    pltpu.sync_copy(dst_hbm.at[pl.ds(base, EPT)], dst_all)
    pltpu.sync_copy(w_hbm.at[pl.ds(base, EPT)], w_all)

    def vec_body(i, _):
        sl = pl.ds(i * L, L)
        plsc.addupdate_scatter(deg_l, [dst_all[sl]], w_all[sl])
        return 0

    lax.fori_loop(0, EPT // L, vec_body, 0)
    pltpu.sync_copy(deg_l, out_hbm.at[wid])


def _agg_body(packed_hbm, h_hbm, zeros_hbm, out_hbm,
              bank0, bank1, bank2, rows0, rows1, rows2, acc_sh,
              gs0, gs1, gs2, ss0, ss1, ss2):
    c = lax.axis_index("c")
    s = lax.axis_index("s")
    wid = c * NS + s
    bank = (bank0, bank1, bank2)
    rows = (rows0, rows1, rows2)
    gs = (gs0, gs1, gs2)
    ss = (ss0, ss1, ss2)

    rbase = wid * NCHUNK
    pltpu.sync_copy(zeros_hbm.at[pl.ds(s * RPT, RPT)],
                    acc_sh.at[pl.ds(s * RPT, RPT)])
    plsc.subcore_barrier()

    def stage_and_gather(ci, k):
        pltpu.sync_copy(packed_hbm.at[rbase + ci], bank[k])
        pltpu.async_copy(h_hbm.at[bank[k].at[0]], rows[k], gs[k])

    stage_and_gather(0, 0)
    stage_and_gather(1, 1)

    def process(i, k):
        pltpu.make_async_copy(h_hbm.at[bank[k].at[0]], rows[k], gs[k]).wait()
        rk = rows[k]
        bk = bank[k]

        def scale_group(g, _):
            wv = plsc.bitcast(bk[2, pl.ds(g * L, L)], jnp.float32)
            for t in range(L):
                nv = wv[t]
                e = g * L + t
                for j in range(D // L):
                    sl2 = pl.ds(j * L, L)
                    rk[e, sl2] = rk[e, sl2] * nv
            return 0

        lax.fori_loop(0, CH // L, scale_group, 0)
        pltpu.async_copy(rk, acc_sh.at[bk.at[1]], ss[k], add=True)
        nk = (k + 2) % 3

        @pl.when(i + 2 < NCHUNK)
        def _():
            @pl.when(i >= 1)
            def _():
                pltpu.make_async_copy(rows[nk], acc_sh.at[bank[nk].at[1]],
                                      ss[nk]).wait()
            stage_and_gather(i + 2, nk)

    def triple_body(tr, _):
        process(3 * tr, 0)
        process(3 * tr + 1, 1)
        process(3 * tr + 2, 2)
        return 0

    lax.fori_loop(0, NTRIP, triple_body, 0)
    for k in range(3):
        pltpu.make_async_copy(rows[k], acc_sh.at[bank[k].at[1]], ss[k]).wait()
    plsc.subcore_barrier()
    pltpu.sync_copy(acc_sh.at[pl.ds(s * RPT, RPT)],
                    out_hbm.at[c, pl.ds(s * RPT, RPT)])


@functools.lru_cache(maxsize=None)
def _sc_kernels():
    mesh = plsc.VectorSubcoreMesh(core_axis_name="c", subcore_axis_name="s")
    deg_k = pl.kernel(
        _deg_body,
        out_type=jax.ShapeDtypeStruct((TILES, NP), jnp.float32),
        mesh=mesh,
        scratch_types=[
            pltpu.VMEM((NP,), jnp.float32),
            pltpu.VMEM((EPT,), jnp.int32),
            pltpu.VMEM((EPT,), jnp.float32),
        ],
        compiler_params=pltpu.CompilerParams(needs_layout_passes=False),
    )
    agg_k = pl.kernel(
        _agg_body,
        out_type=jax.ShapeDtypeStruct((NC, NP, D), jnp.float32),
        mesh=mesh,
        scratch_types=[
            pltpu.VMEM((3, CH), jnp.int32),
            pltpu.VMEM((3, CH), jnp.int32),
            pltpu.VMEM((3, CH), jnp.int32),
            pltpu.VMEM((CH, D), jnp.float32),
            pltpu.VMEM((CH, D), jnp.float32),
            pltpu.VMEM((CH, D), jnp.float32),
            pltpu.VMEM_SHARED((NP, D), jnp.float32),
            pltpu.SemaphoreType.DMA,
            pltpu.SemaphoreType.DMA,
            pltpu.SemaphoreType.DMA,
            pltpu.SemaphoreType.DMA,
            pltpu.SemaphoreType.DMA,
            pltpu.SemaphoreType.DMA,
        ],
        compiler_params=pltpu.CompilerParams(needs_layout_passes=False),
    )
    return deg_k, agg_k


def _tc1_body(degp_ref, x_ref, w1_ref, dinv_ref, h1_ref):
    deg = jnp.sum(degp_ref[:, :N], axis=0, keepdims=True)
    dinv_ref[...] = jnp.where(deg > 0, lax.rsqrt(deg), 0.0)
    h1_ref[...] = jnp.dot(x_ref[...], w1_ref[...],
                          preferred_element_type=jnp.float32)


def _bn_relu(accp_ref, dinv_col_ref, b_ref, g_ref, be_ref):
    dc = dinv_col_ref[...]
    a = dc * (accp_ref[0, :N, :] + accp_ref[1, :N, :]) + b_ref[...]
    a = jnp.maximum(a, 0.0)
    m = jnp.mean(a, axis=0, keepdims=True)
    v = jnp.maximum(jnp.mean(a * a, axis=0, keepdims=True) - m * m, 0.0)
    return g_ref[...] * (a - m) * lax.rsqrt(v + 1e-5) + be_ref[...]


def _tc_mid_body(accp_ref, dinv_col_ref, b_ref, g_ref, be_ref, w2_ref,
                 h2_ref):
    h = _bn_relu(accp_ref, dinv_col_ref, b_ref, g_ref, be_ref)
    h2_ref[...] = dinv_col_ref[...] * jnp.dot(
        h, w2_ref[...], preferred_element_type=jnp.float32)


def _tc_post_body(accp_ref, dinv_col_ref, b_ref, g_ref, be_ref, out_ref):
    out_ref[...] = _bn_relu(accp_ref, dinv_col_ref, b_ref, g_ref, be_ref)


_tc1 = pl.pallas_call(
    _tc1_body,
    out_shape=[jax.ShapeDtypeStruct((1, N), jnp.float32),
               jax.ShapeDtypeStruct((N, D), jnp.float32)],
)

_tc_mid = pl.pallas_call(
    _tc_mid_body,
    out_shape=jax.ShapeDtypeStruct((N, D), jnp.float32),
)

_tc_post = pl.pallas_call(
    _tc_post_body,
    out_shape=jax.ShapeDtypeStruct((N, D), jnp.float32),
)


def kernel(x, edge_index, edge_attr, W1, b1, g1, be1, W2, b2, g2, be2):
    src = edge_index[0]
    dst = edge_index[1]
    loop = jnp.arange(N, dtype=jnp.int32)
    padn = E_PAD - E2
    zi = jnp.zeros((padn,), jnp.int32)
    pad_dst = N + (jnp.arange(padn, dtype=jnp.int32) % (NP - N))
    src2 = jnp.concatenate([src, loop, zi])
    dst2 = jnp.concatenate([dst, loop, pad_dst])
    w2_ = jnp.concatenate([edge_attr, jnp.ones((N,), jnp.float32),
                           jnp.zeros((padn,), jnp.float32)])

    packed = jnp.stack([
        src2.reshape(TILES * NCHUNK, CH),
        dst2.reshape(TILES * NCHUNK, CH),
        jax.lax.bitcast_convert_type(w2_, jnp.int32).reshape(
            TILES * NCHUNK, CH),
    ], axis=1)

    deg_k, agg_k = _sc_kernels()
    degp = deg_k(dst2, w2_)
    dinv2d, hw = _tc1(degp, x, W1)
    dinv_col = dinv2d.reshape(N, 1)
    h1p = dinv_col * hw
    zeros_nd = jnp.zeros((NP, D), jnp.float32)

    accp1 = agg_k(packed, h1p, zeros_nd)
    h2p = _tc_mid(accp1, dinv_col, b1.reshape(1, D), g1.reshape(1, D),
                  be1.reshape(1, D), W2)
    accp2 = agg_k(packed, h2p, zeros_nd)
    out = _tc_post(accp2, dinv_col, b2.reshape(1, D), g2.reshape(1, D),
                   be2.reshape(1, D))
    return out


# speedup vs baseline: 3.1935x; 1.0127x over previous
import functools

import jax
import jax.numpy as jnp
from jax import lax
from jax.experimental import pallas as pl
from jax.experimental.pallas import tpu as pltpu
from jax.experimental.pallas import tpu_sc as plsc

N = 10000
E = 320000
D = 128

NC = 2
NS = 16
L = 16
TILES = NC * NS

E2 = E + N
EPT = 10368
E_PAD = EPT * TILES
CH = 96
NCHUNK = EPT // CH
NTRIP = NCHUNK // 3
NP = 10112
RPT = NP // NS
NBLK = NCHUNK // 12


def _deg_body(dst_hbm, w_hbm, out_hbm, deg_l, dst_all, w_all):
    c = lax.axis_index("c")
    s = lax.axis_index("s")
    wid = c * NS + s

    def zbody(i, _):
        deg_l[pl.ds(i * L, L)] = jnp.zeros((L,), jnp.float32)
        return 0

    lax.fori_loop(0, NP // L, zbody, 0)

    base = wid * EPT
    pltpu.sync_copy(dst_hbm.at[pl.ds(base, EPT)], dst_all)
    pltpu.sync_copy(w_hbm.at[pl.ds(base, EPT)], w_all)

    def vec_body(i, _):
        sl = pl.ds(i * L, L)
        plsc.addupdate_scatter(deg_l, [dst_all[sl]], w_all[sl])
        return 0

    lax.fori_loop(0, EPT // L, vec_body, 0)
    pltpu.sync_copy(deg_l, out_hbm.at[wid])


def _agg_body(packed_hbm, h_hbm, zeros_hbm, out_hbm,
              pk0, pk1, pk2, pk3, dstb0, dstb1, dstb2,
              rows0, rows1, rows2, acc_sh,
              ps0, ps1, ps2, ps3, gs0, gs1, gs2, ss0, ss1, ss2):
    c = lax.axis_index("c")
    s = lax.axis_index("s")
    wid = c * NS + s
    pk = (pk0, pk1, pk2, pk3)
    dstb = (dstb0, dstb1, dstb2)
    rows = (rows0, rows1, rows2)
    ps = (ps0, ps1, ps2, ps3)
    gs = (gs0, gs1, gs2)
    ss = (ss0, ss1, ss2)

    rbase = wid * NCHUNK
    pltpu.sync_copy(zeros_hbm.at[pl.ds(s * RPT, RPT)],
                    acc_sh.at[pl.ds(s * RPT, RPT)])
    plsc.subcore_barrier()

    def stage_start(ci, p):
        pltpu.async_copy(packed_hbm.at[rbase + ci], pk[p], ps[p])

    def gather_start(ci, k, p):
        pltpu.make_async_copy(packed_hbm.at[rbase + ci], pk[p], ps[p]).wait()
        pltpu.async_copy(h_hbm.at[pk[p].at[0]], rows[k], gs[k])

    stage_start(0, 0)
    stage_start(1, 1)
    stage_start(2, 2)
    gather_start(0, 0, 0)
    gather_start(1, 1, 1)

    def process(i, k, p):
        pltpu.make_async_copy(h_hbm.at[pk[p].at[0]], rows[k], gs[k]).wait()
        rk = rows[k]
        bk = pk[p]
        for g in range(CH // L):
            sl = pl.ds(g * L, L)
            dstb[k][sl] = bk[1, sl]

        def scale_group(g, _):
            wv = plsc.bitcast(bk[2, pl.ds(g * L, L)], jnp.float32)
            for t in range(L):
                nv = wv[t]
                e = g * L + t
                for j in range(D // L):
                    sl2 = pl.ds(j * L, L)
                    rk[e, sl2] = rk[e, sl2] * nv
            return 0

        lax.fori_loop(0, CH // L, scale_group, 0)
        pltpu.async_copy(rk, acc_sh.at[dstb[k]], ss[k], add=True)
        np_ = (p + 3) % 4

        @pl.when(i + 3 < NCHUNK)
        def _():
            stage_start(i + 3, np_)

        nk = (k + 2) % 3

        @pl.when(i + 2 < NCHUNK)
        def _():
            @pl.when(i >= 1)
            def _():
                pltpu.make_async_copy(rows[nk], acc_sh.at[dstb[nk]],
                                      ss[nk]).wait()
            gather_start(i + 2, nk, (p + 2) % 4)

    def block_body(b, _):
        for u in range(12):
            process(12 * b + u, u % 3, u % 4)
        return 0

    lax.fori_loop(0, NBLK, block_body, 0)
    for k in range(3):
        pltpu.make_async_copy(rows[k], acc_sh.at[dstb[k]], ss[k]).wait()
    plsc.subcore_barrier()
    pltpu.sync_copy(acc_sh.at[pl.ds(s * RPT, RPT)],
                    out_hbm.at[c, pl.ds(s * RPT, RPT)])


@functools.lru_cache(maxsize=None)
def _sc_kernels():
    mesh = plsc.VectorSubcoreMesh(core_axis_name="c", subcore_axis_name="s")
    deg_k = pl.kernel(
        _deg_body,
        out_type=jax.ShapeDtypeStruct((TILES, NP), jnp.float32),
        mesh=mesh,
        scratch_types=[
            pltpu.VMEM((NP,), jnp.float32),
            pltpu.VMEM((EPT,), jnp.int32),
            pltpu.VMEM((EPT,), jnp.float32),
        ],
        compiler_params=pltpu.CompilerParams(needs_layout_passes=False),
    )
    agg_k = pl.kernel(
        _agg_body,
        out_type=jax.ShapeDtypeStruct((NC, NP, D), jnp.float32),
        mesh=mesh,
        scratch_types=[
            pltpu.VMEM((3, CH), jnp.int32),
            pltpu.VMEM((3, CH), jnp.int32),
            pltpu.VMEM((3, CH), jnp.int32),
            pltpu.VMEM((3, CH), jnp.int32),
            pltpu.VMEM((CH,), jnp.int32),
            pltpu.VMEM((CH,), jnp.int32),
            pltpu.VMEM((CH,), jnp.int32),
            pltpu.VMEM((CH, D), jnp.float32),
            pltpu.VMEM((CH, D), jnp.float32),
            pltpu.VMEM((CH, D), jnp.float32),
            pltpu.VMEM_SHARED((NP, D), jnp.float32),
            pltpu.SemaphoreType.DMA,
            pltpu.SemaphoreType.DMA,
            pltpu.SemaphoreType.DMA,
            pltpu.SemaphoreType.DMA,
            pltpu.SemaphoreType.DMA,
            pltpu.SemaphoreType.DMA,
            pltpu.SemaphoreType.DMA,
            pltpu.SemaphoreType.DMA,
            pltpu.SemaphoreType.DMA,
            pltpu.SemaphoreType.DMA,
        ],
        compiler_params=pltpu.CompilerParams(needs_layout_passes=False),
    )
    return deg_k, agg_k


def _tc1_body(degp_ref, x_ref, w1_ref, dinv_ref, h1_ref):
    deg = jnp.sum(degp_ref[:, :N], axis=0, keepdims=True)
    dinv_ref[...] = jnp.where(deg > 0, lax.rsqrt(deg), 0.0)
    h1_ref[...] = jnp.dot(x_ref[...], w1_ref[...],
                          preferred_element_type=jnp.float32)


def _bn_relu(accp_ref, dinv_col_ref, b_ref, g_ref, be_ref):
    dc = dinv_col_ref[...]
    a = dc * (accp_ref[0, :N, :] + accp_ref[1, :N, :]) + b_ref[...]
    a = jnp.maximum(a, 0.0)
    m = jnp.mean(a, axis=0, keepdims=True)
    v = jnp.maximum(jnp.mean(a * a, axis=0, keepdims=True) - m * m, 0.0)
    return g_ref[...] * (a - m) * lax.rsqrt(v + 1e-5) + be_ref[...]


def _tc_mid_body(accp_ref, dinv_col_ref, b_ref, g_ref, be_ref, w2_ref,
                 h2_ref):
    h = _bn_relu(accp_ref, dinv_col_ref, b_ref, g_ref, be_ref)
    h2_ref[...] = dinv_col_ref[...] * jnp.dot(
        h, w2_ref[...], preferred_element_type=jnp.float32)


def _tc_post_body(accp_ref, dinv_col_ref, b_ref, g_ref, be_ref, out_ref):
    out_ref[...] = _bn_relu(accp_ref, dinv_col_ref, b_ref, g_ref, be_ref)


_tc1 = pl.pallas_call(
    _tc1_body,
    out_shape=[jax.ShapeDtypeStruct((1, N), jnp.float32),
               jax.ShapeDtypeStruct((N, D), jnp.float32)],
)

_tc_mid = pl.pallas_call(
    _tc_mid_body,
    out_shape=jax.ShapeDtypeStruct((N, D), jnp.float32),
)

_tc_post = pl.pallas_call(
    _tc_post_body,
    out_shape=jax.ShapeDtypeStruct((N, D), jnp.float32),
)


def kernel(x, edge_index, edge_attr, W1, b1, g1, be1, W2, b2, g2, be2):
    src = edge_index[0]
    dst = edge_index[1]
    loop = jnp.arange(N, dtype=jnp.int32)
    padn = E_PAD - E2
    zi = jnp.zeros((padn,), jnp.int32)
    pad_dst = N + (jnp.arange(padn, dtype=jnp.int32) % (NP - N))
    src2 = jnp.concatenate([src, loop, zi])
    dst2 = jnp.concatenate([dst, loop, pad_dst])
    w2_ = jnp.concatenate([edge_attr, jnp.ones((N,), jnp.float32),
                           jnp.zeros((padn,), jnp.float32)])

    packed = jnp.stack([
        src2.reshape(TILES * NCHUNK, CH),
        dst2.reshape(TILES * NCHUNK, CH),
        jax.lax.bitcast_convert_type(w2_, jnp.int32).reshape(
            TILES * NCHUNK, CH),
    ], axis=1)

    deg_k, agg_k = _sc_kernels()
    degp = deg_k(dst2, w2_)
    dinv2d, hw = _tc1(degp, x, W1)
    dinv_col = dinv2d.reshape(N, 1)
    h1p = dinv_col * hw
    zeros_nd = jnp.zeros((NP, D), jnp.float32)

    accp1 = agg_k(packed, h1p, zeros_nd)
    h2p = _tc_mid(accp1, dinv_col, b1.reshape(1, D), g1.reshape(1, D),
                  be1.reshape(1, D), W2)
    accp2 = agg_k(packed, h2p, zeros_nd)
    out = _tc_post(accp2, dinv_col, b2.reshape(1, D), g2.reshape(1, D),
                   be2.reshape(1, D))
    return out


# speedup vs baseline: 3.4003x; 1.0648x over previous
import functools

import jax
import jax.numpy as jnp
from jax import lax
from jax.experimental import pallas as pl
from jax.experimental.pallas import tpu as pltpu
from jax.experimental.pallas import tpu_sc as plsc

N = 10000
E = 320000
D = 128

NC = 2
NS = 16
L = 16
TILES = NC * NS

E2 = E + N
EPT = 10368
E_PAD = EPT * TILES
CH = 96
NCHUNK = EPT // CH
NTRIP = NCHUNK // 3
NP = 10112
RPT = NP // NS
NBLK = NCHUNK // 12


def _deg_body(dst_hbm, w_hbm, out_hbm, deg_l, dst_all, w_all):
    c = lax.axis_index("c")
    s = lax.axis_index("s")
    wid = c * NS + s

    def zbody(i, _):
        deg_l[pl.ds(i * L, L)] = jnp.zeros((L,), jnp.float32)
        return 0

    lax.fori_loop(0, NP // L, zbody, 0)

    base = wid * EPT
    pltpu.sync_copy(dst_hbm.at[pl.ds(base, EPT)], dst_all)
    pltpu.sync_copy(w_hbm.at[pl.ds(base, EPT)], w_all)

    def vec_body(i, _):
        sl = pl.ds(i * L, L)
        plsc.addupdate_scatter(deg_l, [dst_all[sl]], w_all[sl])
        return 0

    lax.fori_loop(0, EPT // L, vec_body, 0)
    pltpu.sync_copy(deg_l, out_hbm.at[wid])


def _agg_body(packed_hbm, h_hbm, zeros_hbm, out_hbm,
              pk0, pk1, pk2, pk3, dstb0, dstb1, dstb2,
              rows0, rows1, rows2, acc_sh,
              ps0, ps1, ps2, ps3, gs0, gs1, gs2, ss0, ss1, ss2, zs):
    c = lax.axis_index("c")
    s = lax.axis_index("s")
    wid = c * NS + s
    pk = (pk0, pk1, pk2, pk3)
    dstb = (dstb0, dstb1, dstb2)
    rows = (rows0, rows1, rows2)
    ps = (ps0, ps1, ps2, ps3)
    gs = (gs0, gs1, gs2)
    ss = (ss0, ss1, ss2)

    def crow(ci):
        return ci * TILES + wid

    pltpu.async_copy(zeros_hbm.at[pl.ds(s * RPT, RPT)],
                     acc_sh.at[pl.ds(s * RPT, RPT)], zs)

    def stage_start(ci, p):
        pltpu.async_copy(packed_hbm.at[crow(ci)], pk[p], ps[p])

    def gather_start(ci, k, p):
        pltpu.make_async_copy(packed_hbm.at[crow(ci)], pk[p], ps[p]).wait()
        pltpu.async_copy(h_hbm.at[pk[p].at[0]], rows[k], gs[k])

    stage_start(0, 0)
    stage_start(1, 1)
    stage_start(2, 2)
    gather_start(0, 0, 0)
    gather_start(1, 1, 1)
    pltpu.make_async_copy(zeros_hbm.at[pl.ds(s * RPT, RPT)],
                          acc_sh.at[pl.ds(s * RPT, RPT)], zs).wait()
    plsc.subcore_barrier()

    def process(i, k, p):
        pltpu.make_async_copy(h_hbm.at[pk[p].at[0]], rows[k], gs[k]).wait()
        rk = rows[k]
        bk = pk[p]
        for g in range(CH // L):
            sl = pl.ds(g * L, L)
            dstb[k][sl] = bk[1, sl]

        def scale_group(g, _):
            wv = plsc.bitcast(bk[2, pl.ds(g * L, L)], jnp.float32)
            for t in range(L):
                nv = wv[t]
                e = g * L + t
                for j in range(D // L):
                    sl2 = pl.ds(j * L, L)
                    rk[e, sl2] = rk[e, sl2] * nv
            return 0

        lax.fori_loop(0, CH // L, scale_group, 0)
        pltpu.async_copy(rk, acc_sh.at[dstb[k]], ss[k], add=True)
        np_ = (p + 3) % 4

        @pl.when(i + 3 < NCHUNK)
        def _():
            stage_start(i + 3, np_)

        nk = (k + 2) % 3

        @pl.when(i + 2 < NCHUNK)
        def _():
            @pl.when(i >= 1)
            def _():
                pltpu.make_async_copy(rows[nk], acc_sh.at[dstb[nk]],
                                      ss[nk]).wait()
            gather_start(i + 2, nk, (p + 2) % 4)

    def block_body(b, _):
        for u in range(12):
            process(12 * b + u, u % 3, u % 4)
        return 0

    lax.fori_loop(0, NBLK, block_body, 0)
    for k in range(3):
        pltpu.make_async_copy(rows[k], acc_sh.at[dstb[k]], ss[k]).wait()
    plsc.subcore_barrier()
    pltpu.sync_copy(acc_sh.at[pl.ds(s * RPT, RPT)],
                    out_hbm.at[c, pl.ds(s * RPT, RPT)])


@functools.lru_cache(maxsize=None)
def _sc_kernels():
    mesh = plsc.VectorSubcoreMesh(core_axis_name="c", subcore_axis_name="s")
    deg_k = pl.kernel(
        _deg_body,
        out_type=jax.ShapeDtypeStruct((TILES, NP), jnp.float32),
        mesh=mesh,
        scratch_types=[
            pltpu.VMEM((NP,), jnp.float32),
            pltpu.VMEM((EPT,), jnp.int32),
            pltpu.VMEM((EPT,), jnp.float32),
        ],
        compiler_params=pltpu.CompilerParams(needs_layout_passes=False),
    )
    agg_k = pl.kernel(
        _agg_body,
        out_type=jax.ShapeDtypeStruct((NC, NP, D), jnp.float32),
        mesh=mesh,
        scratch_types=[
            pltpu.VMEM((3, CH), jnp.int32),
            pltpu.VMEM((3, CH), jnp.int32),
            pltpu.VMEM((3, CH), jnp.int32),
            pltpu.VMEM((3, CH), jnp.int32),
            pltpu.VMEM((CH,), jnp.int32),
            pltpu.VMEM((CH,), jnp.int32),
            pltpu.VMEM((CH,), jnp.int32),
            pltpu.VMEM((CH, D), jnp.float32),
            pltpu.VMEM((CH, D), jnp.float32),
            pltpu.VMEM((CH, D), jnp.float32),
            pltpu.VMEM_SHARED((NP, D), jnp.float32),
            pltpu.SemaphoreType.DMA,
            pltpu.SemaphoreType.DMA,
            pltpu.SemaphoreType.DMA,
            pltpu.SemaphoreType.DMA,
            pltpu.SemaphoreType.DMA,
            pltpu.SemaphoreType.DMA,
            pltpu.SemaphoreType.DMA,
            pltpu.SemaphoreType.DMA,
            pltpu.SemaphoreType.DMA,
            pltpu.SemaphoreType.DMA,
            pltpu.SemaphoreType.DMA,
        ],
        compiler_params=pltpu.CompilerParams(needs_layout_passes=False),
    )
    return deg_k, agg_k


def _tc1_body(degp_ref, x_ref, w1_ref, dinv_ref, h1_ref):
    deg = jnp.sum(degp_ref[:, :N], axis=0, keepdims=True)
    dinv_ref[...] = jnp.where(deg > 0, lax.rsqrt(deg), 0.0)
    h1_ref[...] = jnp.dot(x_ref[...], w1_ref[...],
                          preferred_element_type=jnp.float32)


def _bn_relu(accp_ref, dinv_col_ref, b_ref, g_ref, be_ref):
    dc = dinv_col_ref[...]
    a = dc * (accp_ref[0, :N, :] + accp_ref[1, :N, :]) + b_ref[...]
    a = jnp.maximum(a, 0.0)
    m = jnp.mean(a, axis=0, keepdims=True)
    v = jnp.maximum(jnp.mean(a * a, axis=0, keepdims=True) - m * m, 0.0)
    return g_ref[...] * (a - m) * lax.rsqrt(v + 1e-5) + be_ref[...]


def _tc_mid_body(accp_ref, dinv_col_ref, b_ref, g_ref, be_ref, w2_ref,
                 h2_ref):
    h = _bn_relu(accp_ref, dinv_col_ref, b_ref, g_ref, be_ref)
    h2_ref[...] = dinv_col_ref[...] * jnp.dot(
        h, w2_ref[...], preferred_element_type=jnp.float32)


def _tc_post_body(accp_ref, dinv_col_ref, b_ref, g_ref, be_ref, out_ref):
    out_ref[...] = _bn_relu(accp_ref, dinv_col_ref, b_ref, g_ref, be_ref)


_tc1 = pl.pallas_call(
    _tc1_body,
    out_shape=[jax.ShapeDtypeStruct((1, N), jnp.float32),
               jax.ShapeDtypeStruct((N, D), jnp.float32)],
)

_tc_mid = pl.pallas_call(
    _tc_mid_body,
    out_shape=jax.ShapeDtypeStruct((N, D), jnp.float32),
)

_tc_post = pl.pallas_call(
    _tc_post_body,
    out_shape=jax.ShapeDtypeStruct((N, D), jnp.float32),
)


def kernel(x, edge_index, edge_attr, W1, b1, g1, be1, W2, b2, g2, be2):
    src = edge_index[0]
    dst = edge_index[1]
    loop = jnp.arange(N, dtype=jnp.int32)
    padn = E_PAD - E2
    zi = jnp.zeros((padn,), jnp.int32)
    pad_dst = N + (jnp.arange(padn, dtype=jnp.int32) % (NP - N))
    src2 = jnp.concatenate([src, loop, zi])
    dst2 = jnp.concatenate([dst, loop, pad_dst])
    w2_ = jnp.concatenate([edge_attr, jnp.ones((N,), jnp.float32),
                           jnp.zeros((padn,), jnp.float32)])

    packed = jnp.stack([
        src2.reshape(TILES * NCHUNK, CH),
        dst2.reshape(TILES * NCHUNK, CH),
        jax.lax.bitcast_convert_type(w2_, jnp.int32).reshape(
            TILES * NCHUNK, CH),
    ], axis=1)

    deg_k, agg_k = _sc_kernels()
    degp = deg_k(dst2, w2_)
    dinv2d, hw = _tc1(degp, x, W1)
    dinv_col = dinv2d.reshape(N, 1)
    h1p = dinv_col * hw
    zeros_nd = jnp.zeros((NP, D), jnp.float32)

    accp1 = agg_k(packed, h1p, zeros_nd)
    h2p = _tc_mid(accp1, dinv_col, b1.reshape(1, D), g1.reshape(1, D),
                  be1.reshape(1, D), W2)
    accp2 = agg_k(packed, h2p, zeros_nd)
    out = _tc_post(accp2, dinv_col, b2.reshape(1, D), g2.reshape(1, D),
                   be2.reshape(1, D))
    return out
